# baseline (speedup 1.0000x reference)
import jax
import jax.numpy as jnp
from jax import lax
from jax.experimental import pallas as pl
from jax.experimental.pallas import tpu as pltpu

B, N, D, E = 4, 2048, 4096, 64
CAP = 40
BN1 = 256
NB1 = N // BN1
BN2 = 128
NB2 = N // BN2
EPS = 1e-9


def _route_kernel(x_ref, w_ref, p_ref, cnt_ref, loss_ref, z_ref,
                  c1_ref, c2_ref, pacc_ref, zacc_ref):
    nb = pl.program_id(1)

    @pl.when(nb == 0)
    def _init():
        c1_ref[...] = jnp.zeros_like(c1_ref)
        c2_ref[...] = jnp.zeros_like(c2_ref)
        pacc_ref[...] = jnp.zeros_like(pacc_ref)
        zacc_ref[...] = jnp.zeros_like(zacc_ref)

    @pl.when((pl.program_id(0) == 0) & (nb == 0))
    def _init_scalars():
        loss_ref[...] = jnp.zeros_like(loss_ref)
        z_ref[...] = jnp.zeros_like(z_ref)

    x = x_ref[0]
    w = w_ref[...]
    logits = jnp.dot(x, w, preferred_element_type=jnp.float32)
    m = jnp.max(logits, axis=1, keepdims=True)
    ex = jnp.exp(logits - m)
    s = jnp.sum(ex, axis=1, keepdims=True)
    raw = ex / s
    ii = lax.broadcasted_iota(jnp.int32, (BN1, E), 1)
    g1 = jnp.max(raw, axis=1, keepdims=True)
    i1 = jnp.min(jnp.where(raw == g1, ii, E), axis=1, keepdims=True)
    m1 = (ii == i1).astype(jnp.float32)
    raw2 = raw * (1.0 - m1)
    g2 = jnp.max(raw2, axis=1, keepdims=True)
    i2 = jnp.min(jnp.where(raw2 == g2, ii, E), axis=1, keepdims=True)
    m2 = (ii == i2).astype(jnp.float32)
    denom = g1 + g2 + EPS
    g1n = g1 / denom
    g2n = g2 / denom
    ti = lax.broadcasted_iota(jnp.int32, (BN1, BN1), 0)
    tj = lax.broadcasted_iota(jnp.int32, (BN1, BN1), 1)
    ltri = (tj < ti).astype(jnp.float32)
    ex1 = jnp.dot(ltri, m1, preferred_element_type=jnp.float32)
    ex2 = jnp.dot(ltri, m2, preferred_element_type=jnp.float32)
    pos1 = jnp.sum((ex1 + c1_ref[...]) * m1, axis=1, keepdims=True)
    rp2 = jnp.sum((ex2 + c2_ref[...]) * m2, axis=1, keepdims=True)
    c1_ref[...] += jnp.sum(m1, axis=0, keepdims=True)
    c2_ref[...] += jnp.sum(m2, axis=0, keepdims=True)
    pacc_ref[...] += jnp.sum(raw, axis=0, keepdims=True)
    zacc_ref[...] += jnp.sum(m + jnp.log(s), axis=0, keepdims=True)
    zero = jnp.zeros_like(g1)
    p_ref[0] = jnp.concatenate(
        [g1n, g2n, i1.astype(jnp.float32), i2.astype(jnp.float32),
         pos1, rp2, zero, zero], axis=1)

    @pl.when(nb == NB1 - 1)
    def _finalize():
        cnt_ref[0] = jnp.broadcast_to(c1_ref[...], (8, E))
        loss_ref[...] += jnp.sum(pacc_ref[...] * c1_ref[...], axis=1,
                                 keepdims=True) * (float(E) / (B * float(N) * float(N)))
        z_ref[...] += zacc_ref[...] * (1.0 / B)


def _mat_kernel(p_ref, cnt_ref, comb_ref, disp_ref):
    pb = p_ref[0]
    g1 = pb[:, 0:1]
    g2 = pb[:, 1:2]
    i1 = pb[:, 2:3].astype(jnp.int32)
    i2 = pb[:, 3:4].astype(jnp.int32)
    pos1 = pb[:, 4:5]
    rp2 = pb[:, 5:6]
    cnt = jnp.minimum(cnt_ref[0, 0:1, :], float(CAP))
    ii = lax.broadcasted_iota(jnp.int32, (BN2, E), 1)
    m2 = (ii == i2).astype(jnp.float32)
    pos2 = rp2 + jnp.sum(m2 * cnt, axis=1, keepdims=True)
    p1 = pos1.astype(jnp.int32)
    p2 = pos2.astype(jnp.int32)
    e_io = lax.broadcasted_iota(jnp.int32, (BN2, E, CAP), 1)
    c_io = lax.broadcasted_iota(jnp.int32, (BN2, E, CAP), 2)
    comb = (jnp.where((e_io == i1[:, :, None]) & (c_io == p1[:, :, None]),
                      g1[:, :, None], 0.0)
            + jnp.where((e_io == i2[:, :, None]) & (c_io == p2[:, :, None]),
                        g2[:, :, None], 0.0))
    comb_ref[0] = comb
    disp_ref[0] = jnp.where(comb != 0.0, 1.0, 0.0)


def kernel(x, w_gating):
    p, cnt, loss, z = pl.pallas_call(
        _route_kernel,
        grid=(B, NB1),
        in_specs=[
            pl.BlockSpec((1, BN1, D), lambda b, nb: (b, nb, 0)),
            pl.BlockSpec((D, E), lambda b, nb: (0, 0)),
        ],
        out_specs=[
            pl.BlockSpec((1, BN1, 8), lambda b, nb: (b, nb, 0)),
            pl.BlockSpec((1, 8, E), lambda b, nb: (b, 0, 0)),
            pl.BlockSpec((1, 1), lambda b, nb: (0, 0)),
            pl.BlockSpec((1, 1), lambda b, nb: (0, 0)),
        ],
        out_shape=[
            jax.ShapeDtypeStruct((B, N, 8), jnp.float32),
            jax.ShapeDtypeStruct((B, 8, E), jnp.float32),
            jax.ShapeDtypeStruct((1, 1), jnp.float32),
            jax.ShapeDtypeStruct((1, 1), jnp.float32),
        ],
        scratch_shapes=[
            pltpu.VMEM((1, E), jnp.float32),
            pltpu.VMEM((1, E), jnp.float32),
            pltpu.VMEM((1, E), jnp.float32),
            pltpu.VMEM((1, 1), jnp.float32),
        ],
    )(x, w_gating)
    comb, disp = pl.pallas_call(
        _mat_kernel,
        grid=(B, NB2),
        in_specs=[
            pl.BlockSpec((1, BN2, 8), lambda b, nb: (b, nb, 0)),
            pl.BlockSpec((1, 8, E), lambda b, nb: (b, 0, 0)),
        ],
        out_specs=[
            pl.BlockSpec((1, BN2, E, CAP), lambda b, nb: (b, nb, 0, 0)),
            pl.BlockSpec((1, BN2, E, CAP), lambda b, nb: (b, nb, 0, 0)),
        ],
        out_shape=[
            jax.ShapeDtypeStruct((B, N, E, CAP), jnp.float32),
            jax.ShapeDtypeStruct((B, N, E, CAP), jnp.float32),
        ],
    )(p, cnt)
    return disp, comb, loss.reshape(()), z.reshape(())


# speedup vs baseline: 1.0791x; 1.0791x over previous
import jax
import jax.numpy as jnp
from jax import lax
from jax.experimental import pallas as pl
from jax.experimental.pallas import tpu as pltpu

B, N, D, E = 4, 2048, 4096, 64
CAP = 40
BN1 = 512
NB1 = N // BN1
BN2 = 256
NB2 = N // BN2
EPS = 1e-9


def _route_kernel(x_ref, w_ref, p_ref, cnt_ref, loss_ref, z_ref,
                  c1_ref, c2_ref, pacc_ref, zacc_ref):
    nb = pl.program_id(1)

    @pl.when(nb == 0)
    def _init():
        c1_ref[...] = jnp.zeros_like(c1_ref)
        c2_ref[...] = jnp.zeros_like(c2_ref)
        pacc_ref[...] = jnp.zeros_like(pacc_ref)
        zacc_ref[...] = jnp.zeros_like(zacc_ref)

    @pl.when((pl.program_id(0) == 0) & (nb == 0))
    def _init_scalars():
        loss_ref[...] = jnp.zeros_like(loss_ref)
        z_ref[...] = jnp.zeros_like(z_ref)

    x = x_ref[0]
    w = w_ref[...]
    logits = jnp.dot(x, w, preferred_element_type=jnp.float32)
    m = jnp.max(logits, axis=1, keepdims=True)
    ex = jnp.exp(logits - m)
    s = jnp.sum(ex, axis=1, keepdims=True)
    raw = ex / s
    ii = lax.broadcasted_iota(jnp.int32, (BN1, E), 1)
    g1 = jnp.max(raw, axis=1, keepdims=True)
    i1 = jnp.min(jnp.where(raw == g1, ii, E), axis=1, keepdims=True)
    m1 = (ii == i1).astype(jnp.float32)
    raw2 = raw * (1.0 - m1)
    g2 = jnp.max(raw2, axis=1, keepdims=True)
    i2 = jnp.min(jnp.where(raw2 == g2, ii, E), axis=1, keepdims=True)
    m2 = (ii == i2).astype(jnp.float32)
    denom = g1 + g2 + EPS
    g1n = g1 / denom
    g2n = g2 / denom
    ti = lax.broadcasted_iota(jnp.int32, (BN1, BN1), 0)
    tj = lax.broadcasted_iota(jnp.int32, (BN1, BN1), 1)
    ltri = (tj < ti).astype(jnp.float32)
    ex1 = jnp.dot(ltri, m1, preferred_element_type=jnp.float32)
    ex2 = jnp.dot(ltri, m2, preferred_element_type=jnp.float32)
    pos1 = jnp.sum((ex1 + c1_ref[...]) * m1, axis=1, keepdims=True)
    rp2 = jnp.sum((ex2 + c2_ref[...]) * m2, axis=1, keepdims=True)
    c1_ref[...] += jnp.sum(m1, axis=0, keepdims=True)
    c2_ref[...] += jnp.sum(m2, axis=0, keepdims=True)
    pacc_ref[...] += jnp.sum(raw, axis=0, keepdims=True)
    zacc_ref[...] += jnp.sum(m + jnp.log(s), axis=0, keepdims=True)
    t1 = jnp.where(pos1 < float(CAP),
                   i1.astype(jnp.float32) * float(CAP) + pos1, -1.0)
    zero = jnp.zeros_like(g1)
    p_ref[0] = jnp.concatenate(
        [g1n, g2n, t1, i2.astype(jnp.float32), rp2, zero, zero, zero], axis=1)

    @pl.when(nb == NB1 - 1)
    def _finalize():
        cnt_ref[0] = jnp.broadcast_to(c1_ref[...], (8, E))
        loss_ref[...] += jnp.sum(pacc_ref[...] * c1_ref[...], axis=1,
                                 keepdims=True) * (float(E) / (B * float(N) * float(N)))
        z_ref[...] += zacc_ref[...] * (1.0 / B)


def _mat_kernel(k_ref, p_ref, cnt_ref, comb_ref, disp_ref):
    pb = p_ref[0]
    g1 = pb[:, 0:1]
    g2 = pb[:, 1:2]
    t1 = pb[:, 2:3].astype(jnp.int32)
    i2 = pb[:, 3:4].astype(jnp.int32)
    rp2 = pb[:, 4:5]
    cnt = jnp.minimum(cnt_ref[0, 0:1, :], float(CAP))
    ii = lax.broadcasted_iota(jnp.int32, (BN2, E), 1)
    m2 = (ii == i2).astype(jnp.float32)
    pos2 = rp2 + jnp.sum(m2 * cnt, axis=1, keepdims=True)
    t2 = jnp.where(pos2 < float(CAP),
                   i2.astype(jnp.float32) * float(CAP) + pos2, -1.0).astype(jnp.int32)
    k3 = k_ref[...][None, :, :]
    hit1 = k3 == t1[:, :, None]
    hit2 = k3 == t2[:, :, None]
    comb = (jnp.where(hit1, g1[:, :, None], 0.0)
            + jnp.where(hit2, g2[:, :, None], 0.0))
    comb_ref[0] = comb
    disp_ref[0] = jnp.where(comb != 0.0, 1.0, 0.0)


def kernel(x, w_gating):
    p, cnt, loss, z = pl.pallas_call(
        _route_kernel,
        grid=(B, NB1),
        in_specs=[
            pl.BlockSpec((1, BN1, D), lambda b, nb: (b, nb, 0)),
            pl.BlockSpec((D, E), lambda b, nb: (0, 0)),
        ],
        out_specs=[
            pl.BlockSpec((1, BN1, 8), lambda b, nb: (b, nb, 0)),
            pl.BlockSpec((1, 8, E), lambda b, nb: (b, 0, 0)),
            pl.BlockSpec((1, 1), lambda b, nb: (0, 0)),
            pl.BlockSpec((1, 1), lambda b, nb: (0, 0)),
        ],
        out_shape=[
            jax.ShapeDtypeStruct((B, N, 8), jnp.float32),
            jax.ShapeDtypeStruct((B, 8, E), jnp.float32),
            jax.ShapeDtypeStruct((1, 1), jnp.float32),
            jax.ShapeDtypeStruct((1, 1), jnp.float32),
        ],
        scratch_shapes=[
            pltpu.VMEM((1, E), jnp.float32),
            pltpu.VMEM((1, E), jnp.float32),
            pltpu.VMEM((1, E), jnp.float32),
            pltpu.VMEM((1, 1), jnp.float32),
        ],
    )(x, w_gating)
    kmap = (jnp.arange(E, dtype=jnp.int32)[:, None] * CAP
            + jnp.arange(CAP, dtype=jnp.int32)[None, :])
    comb, disp = pl.pallas_call(
        _mat_kernel,
        grid=(B, NB2),
        in_specs=[
            pl.BlockSpec((E, CAP), lambda b, nb: (0, 0)),
            pl.BlockSpec((1, BN2, 8), lambda b, nb: (b, nb, 0)),
            pl.BlockSpec((1, 8, E), lambda b, nb: (b, 0, 0)),
        ],
        out_specs=[
            pl.BlockSpec((1, BN2, E, CAP), lambda b, nb: (b, nb, 0, 0)),
            pl.BlockSpec((1, BN2, E, CAP), lambda b, nb: (b, nb, 0, 0)),
        ],
        out_shape=[
            jax.ShapeDtypeStruct((B, N, E, CAP), jnp.float32),
            jax.ShapeDtypeStruct((B, N, E, CAP), jnp.float32),
        ],
    )(kmap, p, cnt)
    return disp, comb, loss.reshape(()), z.reshape(())


# speedup vs baseline: 2.2591x; 2.0936x over previous
import jax
import jax.numpy as jnp
from jax import lax
from jax.experimental import pallas as pl
from jax.experimental.pallas import tpu as pltpu

B, N, D, E = 4, 2048, 4096, 64
CAP = 40
BN1 = 512
NB1 = N // BN1
BN2 = 256
NB2 = N // BN2
EPS = 1e-9


def _route_kernel(x_ref, w_ref, p_ref, cnt_ref, loss_ref, z_ref,
                  c1_ref, c2_ref, pacc_ref, zacc_ref):
    nb = pl.program_id(1)

    @pl.when(nb == 0)
    def _init():
        c1_ref[...] = jnp.zeros_like(c1_ref)
        c2_ref[...] = jnp.zeros_like(c2_ref)
        pacc_ref[...] = jnp.zeros_like(pacc_ref)
        zacc_ref[...] = jnp.zeros_like(zacc_ref)

    @pl.when((pl.program_id(0) == 0) & (nb == 0))
    def _init_scalars():
        loss_ref[...] = jnp.zeros_like(loss_ref)
        z_ref[...] = jnp.zeros_like(z_ref)

    x = x_ref[0]
    w = w_ref[...]
    logits = jnp.dot(x, w, preferred_element_type=jnp.float32)
    m = jnp.max(logits, axis=1, keepdims=True)
    ex = jnp.exp(logits - m)
    s = jnp.sum(ex, axis=1, keepdims=True)
    raw = ex / s
    ii = lax.broadcasted_iota(jnp.int32, (BN1, E), 1)
    g1 = jnp.max(raw, axis=1, keepdims=True)
    i1 = jnp.min(jnp.where(raw == g1, ii, E), axis=1, keepdims=True)
    m1 = (ii == i1).astype(jnp.float32)
    raw2 = raw * (1.0 - m1)
    g2 = jnp.max(raw2, axis=1, keepdims=True)
    i2 = jnp.min(jnp.where(raw2 == g2, ii, E), axis=1, keepdims=True)
    m2 = (ii == i2).astype(jnp.float32)
    denom = g1 + g2 + EPS
    g1n = g1 / denom
    g2n = g2 / denom
    ti = lax.broadcasted_iota(jnp.int32, (BN1, BN1), 0)
    tj = lax.broadcasted_iota(jnp.int32, (BN1, BN1), 1)
    ltri = (tj < ti).astype(jnp.float32)
    ex1 = jnp.dot(ltri, m1, preferred_element_type=jnp.float32)
    ex2 = jnp.dot(ltri, m2, preferred_element_type=jnp.float32)
    pos1 = jnp.sum((ex1 + c1_ref[...]) * m1, axis=1, keepdims=True)
    rp2 = jnp.sum((ex2 + c2_ref[...]) * m2, axis=1, keepdims=True)
    c1_ref[...] += jnp.sum(m1, axis=0, keepdims=True)
    c2_ref[...] += jnp.sum(m2, axis=0, keepdims=True)
    pacc_ref[...] += jnp.sum(raw, axis=0, keepdims=True)
    zacc_ref[...] += jnp.sum(m + jnp.log(s), axis=0, keepdims=True)
    t1 = jnp.where(pos1 < float(CAP),
                   i1.astype(jnp.float32) * float(CAP) + pos1, -1.0)
    zero = jnp.zeros_like(g1)
    p_ref[0] = jnp.concatenate(
        [g1n, g2n, t1, i2.astype(jnp.float32), rp2, zero, zero, zero], axis=1)

    @pl.when(nb == NB1 - 1)
    def _finalize():
        cnt_ref[0] = jnp.broadcast_to(c1_ref[...], (8, E))
        loss_ref[...] += jnp.sum(pacc_ref[...] * c1_ref[...], axis=1,
                                 keepdims=True) * (float(E) / (B * float(N) * float(N)))
        z_ref[...] += zacc_ref[...] * (1.0 / B)


def _mat_kernel(p_ref, cnt_ref, comb_ref, disp_ref):
    pb = p_ref[0]
    g1 = pb[:, 0:1]
    g2 = pb[:, 1:2]
    t1 = pb[:, 2:3].astype(jnp.int32)
    i2 = pb[:, 3:4].astype(jnp.int32)
    rp2 = pb[:, 4:5]
    cnt = jnp.minimum(cnt_ref[0, 0:1, :], float(CAP))
    ii = lax.broadcasted_iota(jnp.int32, (BN2, E), 1)
    m2 = (ii == i2).astype(jnp.float32)
    pos2 = rp2 + jnp.sum(m2 * cnt, axis=1, keepdims=True)
    t2 = jnp.where(pos2 < float(CAP),
                   i2.astype(jnp.float32) * float(CAP) + pos2, -1.0).astype(jnp.int32)
    k2 = lax.broadcasted_iota(jnp.int32, (BN2, E * CAP), 1)
    comb = jnp.where(k2 == t1, g1, 0.0) + jnp.where(k2 == t2, g2, 0.0)
    comb_ref[0] = comb
    disp_ref[0] = jnp.where(comb != 0.0, 1.0, 0.0)


def kernel(x, w_gating):
    p, cnt, loss, z = pl.pallas_call(
        _route_kernel,
        grid=(B, NB1),
        in_specs=[
            pl.BlockSpec((1, BN1, D), lambda b, nb: (b, nb, 0)),
            pl.BlockSpec((D, E), lambda b, nb: (0, 0)),
        ],
        out_specs=[
            pl.BlockSpec((1, BN1, 8), lambda b, nb: (b, nb, 0)),
            pl.BlockSpec((1, 8, E), lambda b, nb: (b, 0, 0)),
            pl.BlockSpec((1, 1), lambda b, nb: (0, 0)),
            pl.BlockSpec((1, 1), lambda b, nb: (0, 0)),
        ],
        out_shape=[
            jax.ShapeDtypeStruct((B, N, 8), jnp.float32),
            jax.ShapeDtypeStruct((B, 8, E), jnp.float32),
            jax.ShapeDtypeStruct((1, 1), jnp.float32),
            jax.ShapeDtypeStruct((1, 1), jnp.float32),
        ],
        scratch_shapes=[
            pltpu.VMEM((1, E), jnp.float32),
            pltpu.VMEM((1, E), jnp.float32),
            pltpu.VMEM((1, E), jnp.float32),
            pltpu.VMEM((1, 1), jnp.float32),
        ],
    )(x, w_gating)
    comb, disp = pl.pallas_call(
        _mat_kernel,
        grid=(B, NB2),
        in_specs=[
            pl.BlockSpec((1, BN2, 8), lambda b, nb: (b, nb, 0)),
            pl.BlockSpec((1, 8, E), lambda b, nb: (b, 0, 0)),
        ],
        out_specs=[
            pl.BlockSpec((1, BN2, E * CAP), lambda b, nb: (b, nb, 0)),
            pl.BlockSpec((1, BN2, E * CAP), lambda b, nb: (b, nb, 0)),
        ],
        out_shape=[
            jax.ShapeDtypeStruct((B, N, E * CAP), jnp.float32),
            jax.ShapeDtypeStruct((B, N, E * CAP), jnp.float32),
        ],
    )(p, cnt)
    return (disp.reshape(B, N, E, CAP), comb.reshape(B, N, E, CAP),
            loss.reshape(()), z.reshape(()))


# speedup vs baseline: 2.2782x; 1.0085x over previous
import jax
import jax.numpy as jnp
from jax import lax
from jax.experimental import pallas as pl
from jax.experimental.pallas import tpu as pltpu

B, N, D, E = 4, 2048, 4096, 64
CAP = 40
BN1 = 512
NB1 = N // BN1
BN2 = 512
NB2 = N // BN2
EPS = 1e-9


def _route_kernel(x_ref, w_ref, p_ref, cnt_ref, loss_ref, z_ref,
                  c1_ref, c2_ref, pacc_ref, zacc_ref):
    nb = pl.program_id(1)

    @pl.when(nb == 0)
    def _init():
        c1_ref[...] = jnp.zeros_like(c1_ref)
        c2_ref[...] = jnp.zeros_like(c2_ref)
        pacc_ref[...] = jnp.zeros_like(pacc_ref)
        zacc_ref[...] = jnp.zeros_like(zacc_ref)

    @pl.when((pl.program_id(0) == 0) & (nb == 0))
    def _init_scalars():
        loss_ref[...] = jnp.zeros_like(loss_ref)
        z_ref[...] = jnp.zeros_like(z_ref)

    x = x_ref[0]
    w = w_ref[...]
    logits = jnp.dot(x, w, preferred_element_type=jnp.float32)
    m = jnp.max(logits, axis=1, keepdims=True)
    ex = jnp.exp(logits - m)
    s = jnp.sum(ex, axis=1, keepdims=True)
    raw = ex / s
    ii = lax.broadcasted_iota(jnp.int32, (BN1, E), 1)
    g1 = jnp.max(raw, axis=1, keepdims=True)
    i1 = jnp.min(jnp.where(raw == g1, ii, E), axis=1, keepdims=True)
    m1 = (ii == i1).astype(jnp.float32)
    raw2 = raw * (1.0 - m1)
    g2 = jnp.max(raw2, axis=1, keepdims=True)
    i2 = jnp.min(jnp.where(raw2 == g2, ii, E), axis=1, keepdims=True)
    m2 = (ii == i2).astype(jnp.float32)
    denom = g1 + g2 + EPS
    g1n = g1 / denom
    g2n = g2 / denom
    ti = lax.broadcasted_iota(jnp.int32, (BN1, BN1), 0)
    tj = lax.broadcasted_iota(jnp.int32, (BN1, BN1), 1)
    ltri = (tj < ti).astype(jnp.float32)
    ex1 = jnp.dot(ltri, m1, preferred_element_type=jnp.float32)
    ex2 = jnp.dot(ltri, m2, preferred_element_type=jnp.float32)
    pos1 = jnp.sum((ex1 + c1_ref[...]) * m1, axis=1, keepdims=True)
    rp2 = jnp.sum((ex2 + c2_ref[...]) * m2, axis=1, keepdims=True)
    c1_ref[...] += jnp.sum(m1, axis=0, keepdims=True)
    c2_ref[...] += jnp.sum(m2, axis=0, keepdims=True)
    pacc_ref[...] += jnp.sum(raw, axis=0, keepdims=True)
    zacc_ref[...] += jnp.sum(m + jnp.log(s), axis=0, keepdims=True)
    t1 = jnp.where(pos1 < float(CAP),
                   i1.astype(jnp.float32) * float(CAP) + pos1, -1.0)
    zero = jnp.zeros_like(g1)
    p_ref[0] = jnp.concatenate(
        [g1n, g2n, t1, i2.astype(jnp.float32), rp2, zero, zero, zero], axis=1)

    @pl.when(nb == NB1 - 1)
    def _finalize():
        cnt_ref[0] = jnp.broadcast_to(c1_ref[...], (8, E))
        loss_ref[...] += jnp.sum(pacc_ref[...] * c1_ref[...], axis=1,
                                 keepdims=True) * (float(E) / (B * float(N) * float(N)))
        z_ref[...] += zacc_ref[...] * (1.0 / B)


def _mat_kernel(p_ref, cnt_ref, comb_ref, disp_ref):
    pb = p_ref[0]
    g1 = pb[:, 0:1]
    g2 = pb[:, 1:2]
    t1 = pb[:, 2:3].astype(jnp.int32)
    i2 = pb[:, 3:4].astype(jnp.int32)
    rp2 = pb[:, 4:5]
    cnt = jnp.minimum(cnt_ref[0, 0:1, :], float(CAP))
    ii = lax.broadcasted_iota(jnp.int32, (BN2, E), 1)
    m2 = (ii == i2).astype(jnp.float32)
    pos2 = rp2 + jnp.sum(m2 * cnt, axis=1, keepdims=True)
    t2 = jnp.where(pos2 < float(CAP),
                   i2.astype(jnp.float32) * float(CAP) + pos2, -1.0).astype(jnp.int32)
    k2 = lax.broadcasted_iota(jnp.int32, (BN2, E * CAP), 1)
    comb = jnp.where(k2 == t1, g1, 0.0) + jnp.where(k2 == t2, g2, 0.0)
    comb_ref[0] = comb
    disp_ref[0] = jnp.where(comb != 0.0, 1.0, 0.0)


def kernel(x, w_gating):
    p, cnt, loss, z = pl.pallas_call(
        _route_kernel,
        grid=(B, NB1),
        in_specs=[
            pl.BlockSpec((1, BN1, D), lambda b, nb: (b, nb, 0)),
            pl.BlockSpec((D, E), lambda b, nb: (0, 0)),
        ],
        out_specs=[
            pl.BlockSpec((1, BN1, 8), lambda b, nb: (b, nb, 0)),
            pl.BlockSpec((1, 8, E), lambda b, nb: (b, 0, 0)),
            pl.BlockSpec((1, 1), lambda b, nb: (0, 0)),
            pl.BlockSpec((1, 1), lambda b, nb: (0, 0)),
        ],
        out_shape=[
            jax.ShapeDtypeStruct((B, N, 8), jnp.float32),
            jax.ShapeDtypeStruct((B, 8, E), jnp.float32),
            jax.ShapeDtypeStruct((1, 1), jnp.float32),
            jax.ShapeDtypeStruct((1, 1), jnp.float32),
        ],
        scratch_shapes=[
            pltpu.VMEM((1, E), jnp.float32),
            pltpu.VMEM((1, E), jnp.float32),
            pltpu.VMEM((1, E), jnp.float32),
            pltpu.VMEM((1, 1), jnp.float32),
        ],
    )(x, w_gating)
    comb, disp = pl.pallas_call(
        _mat_kernel,
        grid=(B, NB2),
        in_specs=[
            pl.BlockSpec((1, BN2, 8), lambda b, nb: (b, nb, 0)),
            pl.BlockSpec((1, 8, E), lambda b, nb: (b, 0, 0)),
        ],
        out_specs=[
            pl.BlockSpec((1, BN2, E * CAP), lambda b, nb: (b, nb, 0)),
            pl.BlockSpec((1, BN2, E * CAP), lambda b, nb: (b, nb, 0)),
        ],
        out_shape=[
            jax.ShapeDtypeStruct((B, N, E * CAP), jnp.float32),
            jax.ShapeDtypeStruct((B, N, E * CAP), jnp.float32),
        ],
    )(p, cnt)
    return (disp.reshape(B, N, E, CAP), comb.reshape(B, N, E, CAP),
            loss.reshape(()), z.reshape(()))
